# Initial kernel scaffold; baseline (speedup 1.0000x reference)
import jax
import jax.numpy as jnp
from jax.experimental import pallas as pl


def kernel(x, edge_index, W1, b1, W2, b2, Wfc, bfc):
    raise NotImplementedError("write your pallas kernel here")


# speedup vs baseline: 41.2717x; 41.2717x over previous
import functools

import jax
import jax.numpy as jnp
from jax import lax
from jax.experimental import pallas as pl
from jax.experimental.pallas import tpu as pltpu
from jax.experimental.pallas import tpu_sc as plsc

N = 100000
RB = 2048
NBLK = 49
NP = RB * NBLK
E = 3200000
C = 128
K = 16
NCORE = 2
NSUB = 16
NW = NCORE * NSUB
CPW = 800
CHUNKS = NW * CPW
E_PAD = CHUNKS * C
OUTER = CPW // K
SLICE = NP // NSUB

@functools.cache
def _mesh():
    return plsc.VectorSubcoreMesh(core_axis_name="c", subcore_axis_name="s",
                                  num_cores=NCORE, num_subcores=NSUB)


def _wid():
    return lax.axis_index("c") * NSUB + lax.axis_index("s")


def _sc_deg_body(dst_hbm, zeros_hbm, deg_out, idx_v, ones_v, acc, sem):
    cid = lax.axis_index("c")
    sid = lax.axis_index("s")
    pltpu.sync_copy(zeros_hbm, acc.at[pl.ds(sid * SLICE, SLICE)])
    for i in range(C // 16):
        ones_v[pl.ds(i * 16, 16)] = jnp.full((16,), 1.0, jnp.float32)
    plsc.subcore_barrier()

    wid = _wid()

    def outer(i, carry):
        base = wid * CPW + i * K
        pltpu.sync_copy(dst_hbm.at[pl.ds(base, K)], idx_v)
        descs = [
            pltpu.async_copy(ones_v, acc.at[idx_v.at[j]], sem, add=True)
            for j in range(K)
        ]
        for d in descs:
            d.wait()
        return carry

    lax.fori_loop(0, OUTER, outer, 0)
    plsc.subcore_barrier()
    pltpu.sync_copy(acc.at[pl.ds(sid * SLICE, SLICE)],
                    deg_out.at[cid, pl.ds(sid * SLICE, SLICE)])


@functools.cache
def _sc_deg():
    return pl.kernel(
        _sc_deg_body,
        out_type=jax.ShapeDtypeStruct((NCORE, NP), jnp.float32),
        mesh=_mesh(),
        scratch_types=[
            pltpu.VMEM((K, C), jnp.int32),
            pltpu.VMEM((C,), jnp.float32),
            pltpu.VMEM_SHARED((NP,), jnp.float32),
            pltpu.SemaphoreType.DMA,
        ],
        compiler_params=pltpu.CompilerParams(use_tc_tiling_on_sc=False),
    )


def _make_sc_prop(w):
    def body(src_hbm, dst_hbm, tab_hbm, zeros_hbm, s_out,
             idx_s, idx_d, rows, acc, gsem, ssem):
        cid = lax.axis_index("c")
        sid = lax.axis_index("s")
        pltpu.sync_copy(zeros_hbm, acc.at[pl.ds(sid * SLICE, SLICE), :])
        plsc.subcore_barrier()

        wid = _wid()

        def outer(i, carry):
            base = wid * CPW + i * K
            pltpu.sync_copy(src_hbm.at[pl.ds(base, K)], idx_s)
            pltpu.sync_copy(dst_hbm.at[pl.ds(base, K)], idx_d)
            gd = [
                pltpu.async_copy(tab_hbm.at[idx_s.at[j]], rows.at[j], gsem)
                for j in range(K)
            ]
            for d in gd:
                d.wait()
            sd = [
                pltpu.async_copy(rows.at[j], acc.at[idx_d.at[j]], ssem,
                                 add=True)
                for j in range(K)
            ]
            for d in sd:
                d.wait()
            return carry

        lax.fori_loop(0, OUTER, outer, 0)
        plsc.subcore_barrier()
        pltpu.sync_copy(acc.at[pl.ds(sid * SLICE, SLICE), :],
                        s_out.at[cid, pl.ds(sid * SLICE, SLICE), :])

    return pl.kernel(
        body,
        out_type=jax.ShapeDtypeStruct((NCORE, NP, w), jnp.float32),
        mesh=_mesh(),
        scratch_types=[
            pltpu.VMEM((K, C), jnp.int32),
            pltpu.VMEM((K, C), jnp.int32),
            pltpu.VMEM((K, C, w), jnp.float32),
            pltpu.VMEM_SHARED((NP, w), jnp.float32),
            pltpu.SemaphoreType.DMA,
            pltpu.SemaphoreType.DMA,
        ],
        compiler_params=pltpu.CompilerParams(use_tc_tiling_on_sc=False),
    )


_make_sc_prop = functools.cache(_make_sc_prop)


def _prep_body(deg_ref, x_ref, dinv_ref, xp_ref):
    deg = deg_ref[0] + deg_ref[1] + 1.0
    dinv = lax.rsqrt(deg)
    dinv_ref[...] = dinv
    xp_ref[:, :4] = x_ref[...] * dinv
    xp_ref[:, 4:] = jnp.zeros((RB, 4), jnp.float32)


def _tc_prep(deg2, x_pad):
    return pl.pallas_call(
        _prep_body,
        grid=(NBLK,),
        in_specs=[
            pl.BlockSpec((NCORE, RB, 1), lambda i: (0, i, 0)),
            pl.BlockSpec((RB, 4), lambda i: (i, 0)),
        ],
        out_specs=[
            pl.BlockSpec((RB, 1), lambda i: (i, 0)),
            pl.BlockSpec((RB, 8), lambda i: (i, 0)),
        ],
        out_shape=[
            jax.ShapeDtypeStruct((NP, 1), jnp.float32),
            jax.ShapeDtypeStruct((NP, 8), jnp.float32),
        ],
    )(deg2, x_pad)


def _dense1_body(s1_ref, x_ref, dinv_ref, w1_ref, b1_ref, w2_ref,
                 g_ref, gp_ref):
    dinv = dinv_ref[...]
    p1 = (dinv * (s1_ref[0][:, :4] + s1_ref[1][:, :4])
          + (dinv * dinv) * x_ref[...])
    h1 = jnp.maximum(
        jnp.dot(p1, w1_ref[...], preferred_element_type=jnp.float32)
        + b1_ref[...], 0.0)
    g = jnp.dot(h1, w2_ref[...], preferred_element_type=jnp.float32)
    g_ref[...] = g
    gp_ref[...] = dinv * g


def _tc_dense1(s1, x_pad, dinv, W1, b1, W2):
    return pl.pallas_call(
        _dense1_body,
        grid=(NBLK,),
        in_specs=[
            pl.BlockSpec((NCORE, RB, 8), lambda i: (0, i, 0)),
            pl.BlockSpec((RB, 4), lambda i: (i, 0)),
            pl.BlockSpec((RB, 1), lambda i: (i, 0)),
            pl.BlockSpec((4, 16), lambda i: (0, 0)),
            pl.BlockSpec((16,), lambda i: (0,)),
            pl.BlockSpec((16, 8), lambda i: (0, 0)),
        ],
        out_specs=[
            pl.BlockSpec((RB, 8), lambda i: (i, 0)),
            pl.BlockSpec((RB, 8), lambda i: (i, 0)),
        ],
        out_shape=[
            jax.ShapeDtypeStruct((NP, 8), jnp.float32),
            jax.ShapeDtypeStruct((NP, 8), jnp.float32),
        ],
    )(s1, x_pad, dinv, W1, b1, W2)


def _final_body(s2_ref, g_ref, dinv_ref, b2_ref, wfc_ref, bfc_ref, o_ref):
    dinv = dinv_ref[...]
    p2 = dinv * (s2_ref[0] + s2_ref[1]) + (dinv * dinv) * g_ref[...]
    h2 = jnp.maximum(p2 + b2_ref[...], 0.0)
    o_ref[...] = (
        jnp.dot(h2, wfc_ref[...], preferred_element_type=jnp.float32)
        + bfc_ref[...])


def _tc_final(s2, g, dinv, b2, Wfc, bfc):
    return pl.pallas_call(
        _final_body,
        grid=(NBLK,),
        in_specs=[
            pl.BlockSpec((NCORE, RB, 8), lambda i: (0, i, 0)),
            pl.BlockSpec((RB, 8), lambda i: (i, 0)),
            pl.BlockSpec((RB, 1), lambda i: (i, 0)),
            pl.BlockSpec((8,), lambda i: (0,)),
            pl.BlockSpec((8, 1), lambda i: (0, 0)),
            pl.BlockSpec((1,), lambda i: (0,)),
        ],
        out_specs=pl.BlockSpec((RB, 1), lambda i: (i, 0)),
        out_shape=jax.ShapeDtypeStruct((NP, 1), jnp.float32),
    )(s2, g, dinv, b2, Wfc, bfc)


def kernel(x, edge_index, W1, b1, W2, b2, Wfc, bfc):
    x = x.astype(jnp.float32)
    ei = edge_index.astype(jnp.int32)
    pad = jnp.full((E_PAD - E,), N, jnp.int32)
    srcp = jnp.concatenate([ei[0], pad]).reshape(CHUNKS, C)
    dstp = jnp.concatenate([ei[1], pad]).reshape(CHUNKS, C)
    x_pad = jnp.zeros((NP, 4), jnp.float32).at[:N].set(x)
    zeros_d = jnp.zeros((SLICE,), jnp.float32)
    zeros_8 = jnp.zeros((SLICE, 8), jnp.float32)

    deg2 = _sc_deg()(dstp, zeros_d)
    dinv, xp = _tc_prep(deg2.reshape(NCORE, NP, 1), x_pad)
    s1 = _make_sc_prop(8)(srcp, dstp, xp, zeros_8)
    g, gp = _tc_dense1(s1, x_pad, dinv, W1, b1, W2)
    s2 = _make_sc_prop(8)(srcp, dstp, gp, zeros_8)
    o = _tc_final(s2, g, dinv, b2, Wfc, bfc)
    return o[:N, 0]


# speedup vs baseline: 66.0772x; 1.6010x over previous
import functools

import jax
import jax.numpy as jnp
from jax import lax
from jax.experimental import pallas as pl
from jax.experimental.pallas import tpu as pltpu
from jax.experimental.pallas import tpu_sc as plsc

N = 100000
RB = 2048
NBLK = 49
NP = RB * NBLK
E = 3200000
C = 128
K = 16
NCORE = 2
NSUB = 16
NW = NCORE * NSUB
CPW = 800
CHUNKS = NW * CPW
E_PAD = CHUNKS * C
OUTER = CPW // K
SLICE = NP // NSUB

@functools.cache
def _mesh():
    return plsc.VectorSubcoreMesh(core_axis_name="c", subcore_axis_name="s",
                                  num_cores=NCORE, num_subcores=NSUB)


def _wid():
    return lax.axis_index("c") * NSUB + lax.axis_index("s")


def _sc_deg_body(dst_hbm, zeros_hbm, deg_out, idx_v, ones_v, acc, sem):
    cid = lax.axis_index("c")
    sid = lax.axis_index("s")
    pltpu.sync_copy(zeros_hbm, acc.at[pl.ds(sid * SLICE, SLICE)])
    for i in range(C // 16):
        ones_v[pl.ds(i * 16, 16)] = jnp.full((16,), 1.0, jnp.float32)
    plsc.subcore_barrier()

    wid = _wid()

    def outer(i, carry):
        base = wid * CPW + i * K
        pltpu.sync_copy(dst_hbm.at[pl.ds(base, K)], idx_v)
        descs = [
            pltpu.async_copy(ones_v, acc.at[idx_v.at[j]], sem, add=True)
            for j in range(K)
        ]
        for d in descs:
            d.wait()
        return carry

    lax.fori_loop(0, OUTER, outer, 0)
    plsc.subcore_barrier()
    pltpu.sync_copy(acc.at[pl.ds(sid * SLICE, SLICE)],
                    deg_out.at[cid, pl.ds(sid * SLICE, SLICE)])


@functools.cache
def _sc_deg():
    return pl.kernel(
        _sc_deg_body,
        out_type=jax.ShapeDtypeStruct((NCORE, NP), jnp.float32),
        mesh=_mesh(),
        scratch_types=[
            pltpu.VMEM((K, C), jnp.int32),
            pltpu.VMEM((C,), jnp.float32),
            pltpu.VMEM_SHARED((NP,), jnp.float32),
            pltpu.SemaphoreType.DMA,
        ],
        compiler_params=pltpu.CompilerParams(use_tc_tiling_on_sc=False),
    )


def _make_sc_prop(w):
    def body(src_hbm, dst_hbm, tab_hbm, zeros_hbm, s_out,
             idx_s, idx_d, rows, acc, gsem, ssem):
        cid = lax.axis_index("c")
        sid = lax.axis_index("s")
        pltpu.sync_copy(zeros_hbm, acc.at[pl.ds(sid * SLICE, SLICE), :])
        plsc.subcore_barrier()

        wid = _wid()

        def outer(i, carry):
            base = wid * CPW + i * K
            pltpu.sync_copy(src_hbm.at[pl.ds(base, K)], idx_s)
            pltpu.sync_copy(dst_hbm.at[pl.ds(base, K)], idx_d)
            gd = [
                pltpu.async_copy(tab_hbm.at[idx_s.at[j]], rows.at[j], gsem)
                for j in range(K)
            ]
            for d in gd:
                d.wait()
            sd = [
                pltpu.async_copy(rows.at[j], acc.at[idx_d.at[j]], ssem,
                                 add=True)
                for j in range(K)
            ]
            for d in sd:
                d.wait()
            return carry

        lax.fori_loop(0, OUTER, outer, 0)
        plsc.subcore_barrier()
        pltpu.sync_copy(acc.at[pl.ds(sid * SLICE, SLICE), :],
                        s_out.at[cid, pl.ds(sid * SLICE, SLICE), :])

    return pl.kernel(
        body,
        out_type=jax.ShapeDtypeStruct((NCORE, NP, w), jnp.float32),
        mesh=_mesh(),
        scratch_types=[
            pltpu.VMEM((K, C), jnp.int32),
            pltpu.VMEM((K, C), jnp.int32),
            pltpu.VMEM((K, C, w), jnp.float32),
            pltpu.VMEM_SHARED((NP, w), jnp.float32),
            pltpu.SemaphoreType.DMA,
            pltpu.SemaphoreType.DMA,
        ],
        compiler_params=pltpu.CompilerParams(use_tc_tiling_on_sc=False),
    )


_make_sc_prop = functools.cache(_make_sc_prop)


def _prep_body(deg_ref, x_ref, dinv_ref, xp_ref):
    deg = deg_ref[0] + deg_ref[1] + 1.0
    dinv = lax.rsqrt(deg)
    dinv_ref[...] = dinv
    xp_ref[:, :4] = x_ref[...] * dinv
    xp_ref[:, 4:] = jnp.zeros((RB, 4), jnp.float32)


def _tc_prep(deg2, x_pad):
    return pl.pallas_call(
        _prep_body,
        grid=(NBLK,),
        in_specs=[
            pl.BlockSpec((NCORE, RB, 1), lambda i: (0, i, 0)),
            pl.BlockSpec((RB, 4), lambda i: (i, 0)),
        ],
        out_specs=[
            pl.BlockSpec((RB, 1), lambda i: (i, 0)),
            pl.BlockSpec((RB, 8), lambda i: (i, 0)),
        ],
        out_shape=[
            jax.ShapeDtypeStruct((NP, 1), jnp.float32),
            jax.ShapeDtypeStruct((NP, 8), jnp.float32),
        ],
    )(deg2, x_pad)


def _dense1_body(s1_ref, x_ref, dinv_ref, w1_ref, b1_ref, w2_ref,
                 g_ref, gp_ref):
    dinv = dinv_ref[...]
    p1 = (dinv * (s1_ref[0][:, :4] + s1_ref[1][:, :4])
          + (dinv * dinv) * x_ref[...])
    h1 = jnp.maximum(
        jnp.dot(p1, w1_ref[...], preferred_element_type=jnp.float32)
        + b1_ref[...], 0.0)
    g = jnp.dot(h1, w2_ref[...], preferred_element_type=jnp.float32)
    g_ref[...] = g
    gp_ref[...] = dinv * g


def _tc_dense1(s1, x_pad, dinv, W1, b1, W2):
    return pl.pallas_call(
        _dense1_body,
        grid=(NBLK,),
        in_specs=[
            pl.BlockSpec((NCORE, RB, 8), lambda i: (0, i, 0)),
            pl.BlockSpec((RB, 4), lambda i: (i, 0)),
            pl.BlockSpec((RB, 1), lambda i: (i, 0)),
            pl.BlockSpec((4, 16), lambda i: (0, 0)),
            pl.BlockSpec((16,), lambda i: (0,)),
            pl.BlockSpec((16, 8), lambda i: (0, 0)),
        ],
        out_specs=[
            pl.BlockSpec((RB, 8), lambda i: (i, 0)),
            pl.BlockSpec((RB, 8), lambda i: (i, 0)),
        ],
        out_shape=[
            jax.ShapeDtypeStruct((NP, 8), jnp.float32),
            jax.ShapeDtypeStruct((NP, 8), jnp.float32),
        ],
    )(s1, x_pad, dinv, W1, b1, W2)


def _final_body(s2_ref, g_ref, dinv_ref, b2_ref, wfc_ref, bfc_ref, o_ref):
    dinv = dinv_ref[...]
    p2 = dinv * (s2_ref[0] + s2_ref[1]) + (dinv * dinv) * g_ref[...]
    h2 = jnp.maximum(p2 + b2_ref[...], 0.0)
    o_ref[...] = (
        jnp.dot(h2, wfc_ref[...], preferred_element_type=jnp.float32)
        + bfc_ref[...])


def _tc_final(s2, g, dinv, b2, Wfc, bfc):
    return pl.pallas_call(
        _final_body,
        grid=(NBLK,),
        in_specs=[
            pl.BlockSpec((NCORE, RB, 8), lambda i: (0, i, 0)),
            pl.BlockSpec((RB, 8), lambda i: (i, 0)),
            pl.BlockSpec((RB, 1), lambda i: (i, 0)),
            pl.BlockSpec((8,), lambda i: (0,)),
            pl.BlockSpec((8, 1), lambda i: (0, 0)),
            pl.BlockSpec((1,), lambda i: (0,)),
        ],
        out_specs=pl.BlockSpec((RB, 1), lambda i: (i, 0)),
        out_shape=jax.ShapeDtypeStruct((NP, 1), jnp.float32),
    )(s2, g, dinv, b2, Wfc, bfc)


def kernel(x, edge_index, W1, b1, W2, b2, Wfc, bfc):
    x = x.astype(jnp.float32)
    ei = edge_index.astype(jnp.int32)
    pad = N + jnp.arange(E_PAD - E, dtype=jnp.int32) % (NP - N)
    srcp = jnp.concatenate([ei[0], pad]).reshape(CHUNKS, C)
    dstp = jnp.concatenate([ei[1], pad]).reshape(CHUNKS, C)
    x_pad = jnp.zeros((NP, 4), jnp.float32).at[:N].set(x)
    zeros_d = jnp.zeros((SLICE,), jnp.float32)
    zeros_8 = jnp.zeros((SLICE, 8), jnp.float32)

    deg2 = _sc_deg()(dstp, zeros_d)
    dinv, xp = _tc_prep(deg2.reshape(NCORE, NP, 1), x_pad)
    s1 = _make_sc_prop(8)(srcp, dstp, xp, zeros_8)
    g, gp = _tc_dense1(s1, x_pad, dinv, W1, b1, W2)
    s2 = _make_sc_prop(8)(srcp, dstp, gp, zeros_8)
    o = _tc_final(s2, g, dinv, b2, Wfc, bfc)
    return o[:N, 0]


# speedup vs baseline: 75.9777x; 1.1498x over previous
import functools

import jax
import jax.numpy as jnp
from jax import lax
from jax.experimental import pallas as pl
from jax.experimental.pallas import tpu as pltpu
from jax.experimental.pallas import tpu_sc as plsc

N = 100000
RB = 2048
NBLK = 49
NP = RB * NBLK
E = 3200000
C = 128
K = 16
NCORE = 2
NSUB = 16
NW = NCORE * NSUB
CPW = 800
CHUNKS = NW * CPW
E_PAD = CHUNKS * C
OUTER = CPW // K
SLICE = NP // NSUB

@functools.cache
def _mesh():
    return plsc.VectorSubcoreMesh(core_axis_name="c", subcore_axis_name="s",
                                  num_cores=NCORE, num_subcores=NSUB)


def _wid():
    return lax.axis_index("c") * NSUB + lax.axis_index("s")


def _sc_deg_body(dst_hbm, zeros_hbm, deg_out, idx_v, ones_v, acc, sem):
    cid = lax.axis_index("c")
    sid = lax.axis_index("s")
    pltpu.sync_copy(zeros_hbm, acc.at[pl.ds(sid * SLICE, SLICE)])
    for i in range(C // 16):
        ones_v[pl.ds(i * 16, 16)] = jnp.full((16,), 1.0, jnp.float32)
    plsc.subcore_barrier()

    wid = _wid()

    def outer(i, carry):
        base = wid * CPW + i * K
        pltpu.sync_copy(dst_hbm.at[pl.ds(base, K)], idx_v)
        descs = [
            pltpu.async_copy(ones_v, acc.at[idx_v.at[j]], sem, add=True)
            for j in range(K)
        ]
        for d in descs:
            d.wait()
        return carry

    lax.fori_loop(0, OUTER, outer, 0)
    plsc.subcore_barrier()
    pltpu.sync_copy(acc.at[pl.ds(sid * SLICE, SLICE)],
                    deg_out.at[cid, pl.ds(sid * SLICE, SLICE)])


@functools.cache
def _sc_deg():
    return pl.kernel(
        _sc_deg_body,
        out_type=jax.ShapeDtypeStruct((NCORE, NP), jnp.float32),
        mesh=_mesh(),
        scratch_types=[
            pltpu.VMEM((K, C), jnp.int32),
            pltpu.VMEM((C,), jnp.float32),
            pltpu.VMEM_SHARED((NP,), jnp.float32),
            pltpu.SemaphoreType.DMA,
        ],
        compiler_params=pltpu.CompilerParams(use_tc_tiling_on_sc=False),
    )


def _make_sc_prop(w):
    def body(src_hbm, dst_hbm, tab_hbm, zeros_hbm, s_out,
             idx_s, idx_d, rows, acc, lsem, gsem, ssem):
        cid = lax.axis_index("c")
        sid = lax.axis_index("s")
        pltpu.sync_copy(zeros_hbm, acc.at[pl.ds(sid * SLICE, SLICE), :])
        plsc.subcore_barrier()

        wid = _wid()

        def load_idx(blk, b):
            base = wid * CPW + blk * K
            pltpu.async_copy(src_hbm.at[pl.ds(base, K)], idx_s.at[b], lsem)
            pltpu.async_copy(dst_hbm.at[pl.ds(base, K)], idx_d.at[b], lsem)

        def wait_idx(blk, b):
            base = wid * CPW + blk * K
            pltpu.make_async_copy(src_hbm.at[pl.ds(base, K)], idx_s.at[b],
                                  lsem).wait()
            pltpu.make_async_copy(dst_hbm.at[pl.ds(base, K)], idx_d.at[b],
                                  lsem).wait()

        def drain_scatters(b):
            for j in range(K):
                pltpu.make_async_copy(rows.at[b, j],
                                      acc.at[idx_d.at[b, j]], ssem).wait()

        load_idx(0, 0)

        def outer(i2, carry):
            for b in range(2):
                blk = i2 * 2 + b

                @pl.when(blk >= 1)
                def _():
                    drain_scatters(1 - b)

                @pl.when(blk + 1 < OUTER)
                def _():
                    load_idx(blk + 1, 1 - b)

                wait_idx(blk, b)
                for j in range(K):
                    pltpu.async_copy(tab_hbm.at[idx_s.at[b, j]],
                                     rows.at[b, j], gsem)
                for j in range(K):
                    pltpu.make_async_copy(tab_hbm.at[idx_s.at[b, j]],
                                          rows.at[b, j], gsem).wait()
                    pltpu.async_copy(rows.at[b, j], acc.at[idx_d.at[b, j]],
                                     ssem, add=True)
            return carry

        lax.fori_loop(0, OUTER // 2, outer, 0)
        drain_scatters((OUTER - 1) % 2)
        plsc.subcore_barrier()
        pltpu.sync_copy(acc.at[pl.ds(sid * SLICE, SLICE), :],
                        s_out.at[cid, pl.ds(sid * SLICE, SLICE), :])

    return pl.kernel(
        body,
        out_type=jax.ShapeDtypeStruct((NCORE, NP, w), jnp.float32),
        mesh=_mesh(),
        scratch_types=[
            pltpu.VMEM((2, K, C), jnp.int32),
            pltpu.VMEM((2, K, C), jnp.int32),
            pltpu.VMEM((2, K, C, w), jnp.float32),
            pltpu.VMEM_SHARED((NP, w), jnp.float32),
            pltpu.SemaphoreType.DMA,
            pltpu.SemaphoreType.DMA,
            pltpu.SemaphoreType.DMA,
        ],
        compiler_params=pltpu.CompilerParams(use_tc_tiling_on_sc=False),
    )


_make_sc_prop = functools.cache(_make_sc_prop)


def _prep_body(deg_ref, x_ref, dinv_ref, xp_ref):
    deg = deg_ref[0] + deg_ref[1] + 1.0
    dinv = lax.rsqrt(deg)
    dinv_ref[...] = dinv
    xp_ref[:, :4] = x_ref[...] * dinv
    xp_ref[:, 4:] = jnp.zeros((RB, 4), jnp.float32)


def _tc_prep(deg2, x_pad):
    return pl.pallas_call(
        _prep_body,
        grid=(NBLK,),
        in_specs=[
            pl.BlockSpec((NCORE, RB, 1), lambda i: (0, i, 0)),
            pl.BlockSpec((RB, 4), lambda i: (i, 0)),
        ],
        out_specs=[
            pl.BlockSpec((RB, 1), lambda i: (i, 0)),
            pl.BlockSpec((RB, 8), lambda i: (i, 0)),
        ],
        out_shape=[
            jax.ShapeDtypeStruct((NP, 1), jnp.float32),
            jax.ShapeDtypeStruct((NP, 8), jnp.float32),
        ],
    )(deg2, x_pad)


def _dense1_body(s1_ref, x_ref, dinv_ref, w1_ref, b1_ref, w2_ref,
                 g_ref, gp_ref):
    dinv = dinv_ref[...]
    p1 = (dinv * (s1_ref[0][:, :4] + s1_ref[1][:, :4])
          + (dinv * dinv) * x_ref[...])
    h1 = jnp.maximum(
        jnp.dot(p1, w1_ref[...], preferred_element_type=jnp.float32)
        + b1_ref[...], 0.0)
    g = jnp.dot(h1, w2_ref[...], preferred_element_type=jnp.float32)
    g_ref[...] = g
    gp_ref[...] = dinv * g


def _tc_dense1(s1, x_pad, dinv, W1, b1, W2):
    return pl.pallas_call(
        _dense1_body,
        grid=(NBLK,),
        in_specs=[
            pl.BlockSpec((NCORE, RB, 8), lambda i: (0, i, 0)),
            pl.BlockSpec((RB, 4), lambda i: (i, 0)),
            pl.BlockSpec((RB, 1), lambda i: (i, 0)),
            pl.BlockSpec((4, 16), lambda i: (0, 0)),
            pl.BlockSpec((16,), lambda i: (0,)),
            pl.BlockSpec((16, 8), lambda i: (0, 0)),
        ],
        out_specs=[
            pl.BlockSpec((RB, 8), lambda i: (i, 0)),
            pl.BlockSpec((RB, 8), lambda i: (i, 0)),
        ],
        out_shape=[
            jax.ShapeDtypeStruct((NP, 8), jnp.float32),
            jax.ShapeDtypeStruct((NP, 8), jnp.float32),
        ],
    )(s1, x_pad, dinv, W1, b1, W2)


def _final_body(s2_ref, g_ref, dinv_ref, b2_ref, wfc_ref, bfc_ref, o_ref):
    dinv = dinv_ref[...]
    p2 = dinv * (s2_ref[0] + s2_ref[1]) + (dinv * dinv) * g_ref[...]
    h2 = jnp.maximum(p2 + b2_ref[...], 0.0)
    o_ref[...] = (
        jnp.dot(h2, wfc_ref[...], preferred_element_type=jnp.float32)
        + bfc_ref[...])


def _tc_final(s2, g, dinv, b2, Wfc, bfc):
    return pl.pallas_call(
        _final_body,
        grid=(NBLK,),
        in_specs=[
            pl.BlockSpec((NCORE, RB, 8), lambda i: (0, i, 0)),
            pl.BlockSpec((RB, 8), lambda i: (i, 0)),
            pl.BlockSpec((RB, 1), lambda i: (i, 0)),
            pl.BlockSpec((8,), lambda i: (0,)),
            pl.BlockSpec((8, 1), lambda i: (0, 0)),
            pl.BlockSpec((1,), lambda i: (0,)),
        ],
        out_specs=pl.BlockSpec((RB, 1), lambda i: (i, 0)),
        out_shape=jax.ShapeDtypeStruct((NP, 1), jnp.float32),
    )(s2, g, dinv, b2, Wfc, bfc)


def kernel(x, edge_index, W1, b1, W2, b2, Wfc, bfc):
    x = x.astype(jnp.float32)
    ei = edge_index.astype(jnp.int32)
    pad = N + jnp.arange(E_PAD - E, dtype=jnp.int32) % (NP - N)
    srcp = jnp.concatenate([ei[0], pad]).reshape(CHUNKS, C)
    dstp = jnp.concatenate([ei[1], pad]).reshape(CHUNKS, C)
    x_pad = jnp.zeros((NP, 4), jnp.float32).at[:N].set(x)
    zeros_d = jnp.zeros((SLICE,), jnp.float32)
    zeros_8 = jnp.zeros((SLICE, 8), jnp.float32)

    deg2 = _sc_deg()(dstp, zeros_d)
    dinv, xp = _tc_prep(deg2.reshape(NCORE, NP, 1), x_pad)
    s1 = _make_sc_prop(8)(srcp, dstp, xp, zeros_8)
    g, gp = _tc_dense1(s1, x_pad, dinv, W1, b1, W2)
    s2 = _make_sc_prop(8)(srcp, dstp, gp, zeros_8)
    o = _tc_final(s2, g, dinv, b2, Wfc, bfc)
    return o[:N, 0]
